# Initial kernel scaffold; baseline (speedup 1.0000x reference)
import jax
import jax.numpy as jnp
from jax.experimental import pallas as pl


def kernel(x, in_v, in_g, in_b, out_v, out_g, out_b, codebook):
    raise NotImplementedError("write your pallas kernel here")


# speedup vs baseline: 3.0927x; 3.0927x over previous
import jax
import jax.numpy as jnp
from jax.experimental import pallas as pl
from jax.experimental.pallas import tpu as pltpu

_B, _CIN, _S = 8, 512, 1024
_L, _K, _EMB = 4, 1024, 256
_N = _B * _S
_T = 512
_GRID = _N // _T


def _body(xt_ref, win_ref, inb_ref, wout_ref, outb_ref, cnt_ref, cn2_ref,
          cb_ref,
          xqt_ref, idx_ref, logits_ref, loss_ref):
    step = pl.program_id(0)

    @pl.when(step == 0)
    def _init():
        loss_ref[...] = jnp.zeros((1, 1), jnp.float32)

    x_res = xt_ref[...]
    xq_tot = jnp.zeros((_T, _CIN), jnp.float32)
    loss_acc = jnp.float32(0.0)
    idx_cols = []
    iota = jax.lax.broadcasted_iota(jnp.int32, (_T, _K), 1)
    for l in range(_L):
        x_proj = jnp.dot(x_res, win_ref[l],
                         preferred_element_type=jnp.float32) + inb_ref[l]
        rn = jnp.sqrt(jnp.sum(x_proj * x_proj, axis=1, keepdims=True))
        xn = x_proj / jnp.maximum(rn, 1e-12)
        sim = jnp.dot(xn, cnt_ref[l],
                      preferred_element_type=jnp.float32)
        rn2 = jnp.sum(xn * xn, axis=1, keepdims=True)
        logits = (2.0 * sim - rn2) - cn2_ref[l]
        logits_ref[:, l, :] = logits
        m = jnp.max(logits, axis=1, keepdims=True)
        idx = jnp.min(jnp.where(logits >= m, iota, _K), axis=1,
                      keepdims=True)
        idx_cols.append(idx)
        onehot = (iota == idx).astype(jnp.float32)
        xq = jnp.dot(onehot, cb_ref[l],
                     preferred_element_type=jnp.float32)
        d = xq - x_proj
        loss_acc = loss_acc + jnp.sum(d * d)
        xq_st = x_proj + d
        out = jnp.dot(xq_st, wout_ref[l],
                      preferred_element_type=jnp.float32) + outb_ref[l]
        xq_tot = xq_tot + out
        x_res = x_res - out
    xqt_ref[...] = xq_tot
    idx_ref[...] = jnp.concatenate(idx_cols, axis=1)
    loss_ref[...] += jnp.reshape(loss_acc / (_B * _S * _EMB), (1, 1))


def kernel(x, in_v, in_g, in_b, out_v, out_g, out_b, codebook):
    xt = x.transpose(0, 2, 1).reshape(_N, _CIN)

    n_in = jnp.sqrt(jnp.sum(in_v * in_v, axis=2, keepdims=True))
    w_in = in_v * (in_g[:, :, None] / jnp.maximum(n_in, 1e-12))
    n_out = jnp.sqrt(jnp.sum(out_v * out_v, axis=2, keepdims=True))
    w_out = out_v * (out_g[:, :, None] / jnp.maximum(n_out, 1e-12))
    n_cb = jnp.linalg.norm(codebook, axis=2, keepdims=True)
    cn = codebook / jnp.maximum(n_cb, 1e-12)
    cn2 = jnp.sum(cn * cn, axis=2)

    win_t = w_in.transpose(0, 2, 1)
    wout_t = w_out.transpose(0, 2, 1)
    cnt = cn.transpose(0, 2, 1)
    inb = in_b.reshape(_L, 1, _EMB)
    outb = out_b.reshape(_L, 1, _CIN)
    cn2r = cn2.reshape(_L, 1, _K)

    full = lambda *shape: pl.BlockSpec(shape, lambda i: (0,) * len(shape))
    xqt, idxs, logits, loss = pl.pallas_call(
        _body,
        grid=(_GRID,),
        in_specs=[
            pl.BlockSpec((_T, _CIN), lambda i: (i, 0)),
            full(_L, _CIN, _EMB),
            full(_L, 1, _EMB),
            full(_L, _EMB, _CIN),
            full(_L, 1, _CIN),
            full(_L, _EMB, _K),
            full(_L, 1, _K),
            full(_L, _K, _EMB),
        ],
        out_specs=[
            pl.BlockSpec((_T, _CIN), lambda i: (i, 0)),
            pl.BlockSpec((_T, _L), lambda i: (i, 0)),
            pl.BlockSpec((_T, _L, _K), lambda i: (i, 0, 0)),
            pl.BlockSpec((1, 1), lambda i: (0, 0)),
        ],
        out_shape=[
            jax.ShapeDtypeStruct((_N, _CIN), jnp.float32),
            jax.ShapeDtypeStruct((_N, _L), jnp.int32),
            jax.ShapeDtypeStruct((_N, _L, _K), jnp.float32),
            jax.ShapeDtypeStruct((1, 1), jnp.float32),
        ],
    )(xt, win_t, inb, wout_t, outb, cnt, cn2r, codebook)

    xq_total = xqt.reshape(_B, _S, _CIN).transpose(0, 2, 1)
    x_idxs = idxs.reshape(_B, _S, _L).transpose(0, 2, 1)
    l0 = loss[0, 0]
    return (xq_total, x_idxs, logits, l0, l0)
